# baseline (speedup 1.0000x reference)
import functools

import jax
import jax.numpy as jnp
from jax import lax
from jax.experimental import pallas as pl
from jax.experimental.pallas import tpu as pltpu
from jax.experimental.pallas import tpu_sc as plsc

B = 1024
L = 200
EMB = 64
NC, NS = 2, 16
NW = NC * NS
CHUNK = 128
NCH = L * B // CHUNK // NW
NBUF = 5
AHEAD = 2


@functools.partial(
    pl.kernel,
    out_type=jax.ShapeDtypeStruct((L, 8, 8, 8, CHUNK), jnp.float32),
    mesh=plsc.VectorSubcoreMesh(core_axis_name="c", subcore_axis_name="s"),
    compiler_params=pltpu.CompilerParams(
        use_tc_tiling_on_sc=False, needs_layout_passes=False
    ),
    scratch_types=(
        [pltpu.VMEM((NCH, CHUNK), jnp.int32)]
        + [pltpu.VMEM((CHUNK, EMB), jnp.float32) for _ in range(NBUF)]
        + [pltpu.VMEM((EMB, CHUNK), jnp.float32) for _ in range(NBUF)]
        + [pltpu.VMEM((EMB, 16), jnp.float32) for _ in range(NBUF)]
        + [pltpu.SemaphoreType.DMA for _ in range(2 * NBUF)]
    ),
)
def _emb_lookup(idx_hbm, pos_hbm, table_hbm, out_hbm, *refs):
    idx_v = refs[0]
    rows = refs[1:1 + NBUF]
    rowst = refs[1 + NBUF:1 + 2 * NBUF]
    posc = refs[1 + 2 * NBUF:1 + 3 * NBUF]
    sem_g = refs[1 + 3 * NBUF:1 + 4 * NBUF]
    sem_w = refs[1 + 4 * NBUF:1 + 5 * NBUF]

    wid = lax.axis_index("s") * NC + lax.axis_index("c")
    gbase = wid * NCH
    pltpu.sync_copy(idx_hbm.at[pl.ds(gbase, NCH)], idx_v)

    def chunk_l(c):
        g = gbase + c
        return (g // 64) * 8 + g % 8

    def start_fetch(cn, bn):
        pltpu.async_copy(table_hbm.at[idx_v.at[cn]], rows[bn], sem_g[bn])
        pltpu.async_copy(pos_hbm.at[chunk_l(cn)], posc[bn], sem_g[bn])

    for b in range(AHEAD):
        start_fetch(b, b)

    rid = [lax.iota(jnp.int32, 16) + 16 * blk for blk in range(8)]

    def outer(t, carry):
        for b in range(NBUF):
            c = t * NBUF + b
            pltpu.make_async_copy(
                table_hbm.at[idx_v.at[c]], rows[b], sem_g[b]
            ).wait()
            pltpu.make_async_copy(
                pos_hbm.at[chunk_l(c)], posc[b], sem_g[b]
            ).wait()

            @pl.when(t > 0)
            def _drain(b=b):
                for te in range(8):
                    pltpu.make_async_copy(
                        rowst[b].at[pl.ds(8 * te, 8)],
                        out_hbm.at[0, te, 0],
                        sem_w[b],
                    ).wait()

            def e_body(e, carry2, b=b):
                ecol = jnp.zeros((16,), jnp.int32) + e
                pv = posc[b][e, :]
                for blk in range(8):
                    v = plsc.load_gather(rows[b], [rid[blk], ecol])
                    rowst[b][e, pl.ds(16 * blk, 16)] = v + pv
                return carry2

            lax.fori_loop(0, EMB, e_body, 0)

            g = gbase + c
            l = (g // 64) * 8 + g % 8
            tb = (g // 8) % 8
            for te in range(8):
                pltpu.async_copy(
                    rowst[b].at[pl.ds(8 * te, 8)],
                    out_hbm.at[l, te, tb],
                    sem_w[b],
                )

            bn = (b + AHEAD) % NBUF
            cn = c + AHEAD

            @pl.when(cn < NCH)
            def _prefetch(bn=bn, cn=cn):
                start_fetch(cn, bn)

        return carry

    lax.fori_loop(0, NCH // NBUF, outer, 0)
    for b in range(NBUF):
        for te in range(8):
            pltpu.make_async_copy(
                rowst[b].at[pl.ds(8 * te, 8)], out_hbm.at[0, te, 0], sem_w[b]
            ).wait()


def kernel(x, table, pos_emb):
    xv = (
        x.astype(jnp.int32)
        .T.reshape(L // 8, 8, B // CHUNK, CHUNK)
        .transpose(0, 2, 1, 3)
        .reshape(L * B // CHUNK, CHUNK)
    )
    pos_b = jnp.broadcast_to(
        pos_emb.astype(jnp.float32).reshape(L, EMB)[:, :, None], (L, EMB, 16)
    )
    o5 = _emb_lookup(xv, pos_b, table)
    return o5.transpose(2, 4, 0, 1, 3).reshape(B, L, EMB)


# speedup vs baseline: 1.1977x; 1.1977x over previous
import functools

import jax
import jax.numpy as jnp
from jax import lax
from jax.experimental import pallas as pl
from jax.experimental.pallas import tpu as pltpu
from jax.experimental.pallas import tpu_sc as plsc

B = 1024
L = 200
EMB = 64
N = B * L
NC, NS = 2, 16
NW = NC * NS
CHUNK = 128
NCH = N // CHUNK // NW
NBUF = 5
AHEAD = 3


@functools.partial(
    pl.kernel,
    out_type=jax.ShapeDtypeStruct((N, EMB), jnp.float32),
    mesh=plsc.VectorSubcoreMesh(core_axis_name="c", subcore_axis_name="s"),
    compiler_params=pltpu.CompilerParams(use_tc_tiling_on_sc=False),
    scratch_types=(
        [pltpu.VMEM((NCH, CHUNK), jnp.int32)]
        + [pltpu.VMEM((CHUNK, EMB), jnp.float32) for _ in range(NBUF)]
        + [pltpu.SemaphoreType.DMA for _ in range(2 * NBUF)]
    ),
)
def _emb_gather(idx_hbm, table_hbm, out_hbm, *refs):
    idx_v = refs[0]
    rows = refs[1:1 + NBUF]
    sem_g = refs[1 + NBUF:1 + 2 * NBUF]
    sem_w = refs[1 + 2 * NBUF:1 + 3 * NBUF]

    wid = lax.axis_index("s") * NC + lax.axis_index("c")
    pltpu.sync_copy(idx_hbm.at[wid], idx_v)
    gbase = wid * NCH

    for b in range(AHEAD):
        pltpu.async_copy(table_hbm.at[idx_v.at[b]], rows[b], sem_g[b])

    def outer(t, carry):
        for b in range(NBUF):
            c = t * NBUF + b
            pltpu.make_async_copy(
                table_hbm.at[idx_v.at[c]], rows[b], sem_g[b]
            ).wait()
            pltpu.async_copy(
                rows[b], out_hbm.at[pl.ds((gbase + c) * CHUNK, CHUNK)],
                sem_w[b],
            )

            bn = (b + AHEAD) % NBUF
            cn = c + AHEAD

            @pl.when(cn < NCH)
            def _issue(bn=bn, cn=cn):
                @pl.when(cn >= NBUF)
                def _drain():
                    pltpu.make_async_copy(
                        rows[bn], out_hbm.at[pl.ds(0, CHUNK)], sem_w[bn]
                    ).wait()

                pltpu.async_copy(
                    table_hbm.at[idx_v.at[cn]], rows[bn], sem_g[bn]
                )

        return carry

    lax.fori_loop(0, NCH // NBUF, outer, 0)
    for b in range(NBUF):
        pltpu.make_async_copy(
            rows[b], out_hbm.at[pl.ds(0, CHUNK)], sem_w[b]
        ).wait()


def kernel(x, table, pos_emb):
    idx = x.astype(jnp.int32).T.reshape(NW, NCH, CHUNK)
    out = _emb_gather(idx, table)
    o = out.reshape(L, B // CHUNK, CHUNK, EMB).transpose(1, 2, 0, 3)
    return o.reshape(B, L, EMB) + pos_emb
